# baseline (speedup 1.0000x reference)
import functools

import jax
import jax.numpy as jnp
from jax import lax
from jax.experimental import pallas as pl
from jax.experimental.pallas import tpu as pltpu

B = 4096
D0 = 1024
HID = 1024
E = 8
KTOP = 2
NPAIR = B * KTOP
TM = 128
P = NPAIR + E * TM
NT = P // TM
TN = 512
TMG = 512
NEG = -1e30

_f32 = jnp.float32
_i32 = jnp.int32


def _top2(logits, col):
    valid = col < E
    logits = jnp.where(valid, logits, NEG)
    m = jnp.max(logits, axis=1, keepdims=True)
    p = jnp.exp(logits - m)
    p = jnp.where(valid, p, 0.0)
    s = jnp.sum(p, axis=1, keepdims=True)
    g = p / s
    m1 = jnp.max(g, axis=1, keepdims=True)
    i1 = jnp.min(jnp.where(g == m1, col, 127), axis=1, keepdims=True)
    g2 = jnp.where(col == i1, -1.0, g)
    m2 = jnp.max(g2, axis=1, keepdims=True)
    i2 = jnp.min(jnp.where(g2 == m2, col, 127), axis=1, keepdims=True)
    denom = m1 + m2 + 1e-9
    return i1, i2, m1 / denom, m2 / denom


def _gate_body2(xa_ref, xb_ref, wg_ref, bg_ref, eid_ref, w_ref):
    logits = jnp.dot(xa_ref[...], wg_ref[:D0, :], preferred_element_type=_f32)
    logits += jnp.dot(xb_ref[...], wg_ref[D0:, :], preferred_element_type=_f32)
    logits += bg_ref[...]
    col = lax.broadcasted_iota(_i32, logits.shape, 1)
    i1, i2, w1, w2 = _top2(logits, col)
    eid_ref[...] = jnp.where(col == 0, i1, jnp.where(col == 1, i2, 0)).astype(_i32)
    w_ref[...] = jnp.where(col == 0, w1, jnp.where(col == 1, w2, 0.0))


def _gate_body1(xa_ref, wg_ref, bg_ref, eid_ref, w_ref):
    logits = jnp.dot(xa_ref[...], wg_ref[...], preferred_element_type=_f32)
    logits += bg_ref[...]
    col = lax.broadcasted_iota(_i32, logits.shape, 1)
    i1, i2, w1, w2 = _top2(logits, col)
    eid_ref[...] = jnp.where(col == 0, i1, jnp.where(col == 1, i2, 0)).astype(_i32)
    w_ref[...] = jnp.where(col == 0, w1, jnp.where(col == 1, w2, 0.0))


def _gate(xa, xb, wg, bg):
    din = wg.shape[0]
    wg_pad = jnp.zeros((din, 128), _f32).at[:, :E].set(wg)
    bg_pad = jnp.zeros((1, 128), _f32).at[0, :E].set(bg)
    grid = (B // TMG,)
    row_spec = pl.BlockSpec((TMG, D0), lambda i: (i, 0))
    full = lambda shape: pl.BlockSpec(shape, lambda i: tuple(0 for _ in shape))
    out_specs = [pl.BlockSpec((TMG, 128), lambda i: (i, 0))] * 2
    out_shape = [jax.ShapeDtypeStruct((B, 128), _i32),
                 jax.ShapeDtypeStruct((B, 128), _f32)]
    if xb is None:
        eid, w = pl.pallas_call(
            _gate_body1, grid=grid,
            in_specs=[row_spec, full((din, 128)), full((1, 128))],
            out_specs=out_specs, out_shape=out_shape,
        )(xa, wg_pad, bg_pad)
    else:
        eid, w = pl.pallas_call(
            _gate_body2, grid=grid,
            in_specs=[row_spec, row_spec, full((din, 128)), full((1, 128))],
            out_specs=out_specs, out_shape=out_shape,
        )(xa, xb, wg_pad, bg_pad)
    return eid[:, :KTOP], w[:, :KTOP]


def _route(eid, w):
    ef = eid.reshape(-1).astype(_i32)
    wf = w.reshape(-1)
    onehot = ef[:, None] == jnp.arange(E, dtype=_i32)[None, :]
    counts = onehot.sum(0).astype(_i32)
    padded = ((counts + TM - 1) // TM) * TM
    off = jnp.concatenate([jnp.zeros((1,), _i32), jnp.cumsum(padded)[:-1].astype(_i32)])
    csc = jnp.concatenate([jnp.zeros((1,), _i32), jnp.cumsum(counts)[:-1].astype(_i32)])
    perm = jnp.argsort(ef, stable=True).astype(_i32)
    es = ef[perm]
    k = jnp.arange(NPAIR, dtype=_i32)
    dest_sorted = off[es] + (k - csc[es])
    src_tok = jnp.zeros((P,), _i32).at[dest_sorted].set(perm // KTOP)
    ws = jnp.zeros((P,), _f32).at[dest_sorted].set(wf[perm])
    dest = jnp.zeros((NPAIR,), _i32).at[perm].set(dest_sorted)
    t = jnp.arange(NT, dtype=_i32) * TM
    tile_eid = jnp.clip((t[:, None] >= off[None, :]).sum(1) - 1, 0, E - 1).astype(_i32)
    return src_tok, ws.reshape(P, 1), dest.reshape(B, KTOP), tile_eid


def _gather_rows(x, src_tok):
    return x[src_tok]


def _combine(ys, dest):
    return ys[dest[:, 0]] + ys[dest[:, 1]]


def _bf16r(x):
    return x.astype(jnp.bfloat16).astype(_f32)


def _gmm_body2(eid_ref, xa_ref, xb_ref, w_ref, b_ref, ws_ref, out_ref):
    acc = jnp.dot(xa_ref[...], w_ref[0, :D0, :], preferred_element_type=_f32)
    acc += jnp.dot(xb_ref[...], w_ref[0, D0:, :], preferred_element_type=_f32)
    acc += b_ref[0]
    acc = jnp.where(acc > 0, acc, 0.2 * acc)
    out_ref[...] = _bf16r(acc) * _bf16r(ws_ref[...])


def _gmm_body1(eid_ref, xa_ref, w_ref, b_ref, ws_ref, out_ref):
    acc = jnp.dot(xa_ref[...], w_ref[0], preferred_element_type=_f32)
    acc += b_ref[0]
    acc = jnp.where(acc > 0, acc, 0.2 * acc)
    out_ref[...] = _bf16r(acc) * _bf16r(ws_ref[...])


def _gmm(xa, xb, wexp, bexp, ws, tile_eid):
    din = wexp.shape[1]
    dout = wexp.shape[2]
    nn = dout // TN
    row = lambda j, i, eid: (i, 0)
    in_specs = [pl.BlockSpec((TM, D0), row)]
    args = [xa]
    if xb is not None:
        in_specs.append(pl.BlockSpec((TM, D0), row))
        args.append(xb)
    in_specs += [
        pl.BlockSpec((1, din, TN), lambda j, i, eid: (eid[i], 0, j)),
        pl.BlockSpec((1, 1, TN), lambda j, i, eid: (eid[i], 0, j)),
        pl.BlockSpec((TM, 1), row),
    ]
    args += [wexp, bexp.reshape(E, 1, dout), ws]
    grid_spec = pltpu.PrefetchScalarGridSpec(
        num_scalar_prefetch=1,
        grid=(nn, NT),
        in_specs=in_specs,
        out_specs=pl.BlockSpec((TM, TN), lambda j, i, eid: (i, j)),
    )
    body = _gmm_body1 if xb is None else _gmm_body2
    return pl.pallas_call(
        body, grid_spec=grid_spec,
        out_shape=jax.ShapeDtypeStruct((P, dout), _f32),
    )(tile_eid, *args)


def _final_body(xa_ref, xb_ref, wg_ref, bg_ref, wr_ref, br_ref, out_ref):
    logits = jnp.dot(xa_ref[...], wg_ref[:D0, :], preferred_element_type=_f32)
    logits += jnp.dot(xb_ref[...], wg_ref[D0:, :], preferred_element_type=_f32)
    logits += bg_ref[...]
    col = lax.broadcasted_iota(_i32, logits.shape, 1)
    i1, i2, w1, w2 = _top2(logits, col)
    h = jnp.dot(xa_ref[...], wr_ref[:D0, :], preferred_element_type=_f32)
    h += jnp.dot(xb_ref[...], wr_ref[D0:, :], preferred_element_type=_f32)
    h += br_ref[...]
    h1 = jnp.sum(jnp.where(col == i1, h, 0.0), axis=1, keepdims=True)
    h2 = jnp.sum(jnp.where(col == i2, h, 0.0), axis=1, keepdims=True)
    out = w1 * h1 + w2 * h2
    out_ref[...] = jnp.broadcast_to(out, out_ref.shape)


def _final(xa, xb, p):
    din = p['Wg'].shape[0]
    wg_pad = jnp.zeros((din, 128), _f32).at[:, :E].set(p['Wg'])
    bg_pad = jnp.zeros((1, 128), _f32).at[0, :E].set(p['bg'])
    wr_pad = jnp.zeros((din, 128), _f32).at[:, :E].set(p['W'][:, :, 0].T)
    br_pad = jnp.zeros((1, 128), _f32).at[0, :E].set(p['b'][:, 0])
    grid = (B // TMG,)
    row_spec = pl.BlockSpec((TMG, D0), lambda i: (i, 0))
    full = lambda shape: pl.BlockSpec(shape, lambda i: tuple(0 for _ in shape))
    out = pl.pallas_call(
        _final_body, grid=grid,
        in_specs=[row_spec, row_spec, full((din, 128)), full((1, 128)),
                  full((din, 128)), full((1, 128))],
        out_specs=pl.BlockSpec((TMG, 128), lambda i: (i, 0)),
        out_shape=jax.ShapeDtypeStruct((B, 128), _f32),
    )(xa, xb, wg_pad, bg_pad, wr_pad, br_pad)
    return out[:, :1]


def _layer(prev, x0, p):
    eid, w = _gate(x0 if prev is None else prev, None if prev is None else x0,
                   p['Wg'], p['bg'])
    src_tok, ws, dest, tile_eid = _route(eid, w)
    if prev is None:
        xa = _gather_rows(x0, src_tok)
        ys = _gmm(xa, None, p['W'], p['b'], ws, tile_eid)
    else:
        xa = _gather_rows(prev, src_tok)
        xb = _gather_rows(x0, src_tok)
        ys = _gmm(xa, xb, p['W'], p['b'], ws, tile_eid)
    return _combine(ys, dest)


@jax.jit
def kernel(states, actions, params):
    x0 = jnp.concatenate([states, actions], axis=-1)
    prev = None
    for l in range(4):
        prev = _layer(prev, x0, params['l%d' % l])
    return _final(prev, x0, params['l4'])


# speedup vs baseline: 1.1057x; 1.1057x over previous
import functools

import jax
import jax.numpy as jnp
from jax import lax
from jax.experimental import pallas as pl
from jax.experimental.pallas import tpu as pltpu

B = 4096
D0 = 1024
HID = 1024
E = 8
KTOP = 2
NPAIR = B * KTOP
TM = 128
P = NPAIR + E * TM
NT = P // TM
TN = 512
TMG = 512
NEG = -1e30

_f32 = jnp.float32
_i32 = jnp.int32


def _top2(logits, col):
    valid = col < E
    logits = jnp.where(valid, logits, NEG)
    m = jnp.max(logits, axis=1, keepdims=True)
    p = jnp.exp(logits - m)
    p = jnp.where(valid, p, 0.0)
    s = jnp.sum(p, axis=1, keepdims=True)
    g = p / s
    m1 = jnp.max(g, axis=1, keepdims=True)
    i1 = jnp.min(jnp.where(g == m1, col, 127), axis=1, keepdims=True)
    g2 = jnp.where(col == i1, -1.0, g)
    m2 = jnp.max(g2, axis=1, keepdims=True)
    i2 = jnp.min(jnp.where(g2 == m2, col, 127), axis=1, keepdims=True)
    denom = m1 + m2 + 1e-9
    return i1, i2, m1 / denom, m2 / denom


def _gate_body2(xa_ref, xb_ref, wg_ref, bg_ref, eid_ref, w_ref):
    logits = jnp.dot(xa_ref[...], wg_ref[:D0, :], preferred_element_type=_f32)
    logits += jnp.dot(xb_ref[...], wg_ref[D0:, :], preferred_element_type=_f32)
    logits += bg_ref[...]
    col = lax.broadcasted_iota(_i32, logits.shape, 1)
    i1, i2, w1, w2 = _top2(logits, col)
    eid_ref[...] = jnp.where(col == 0, i1, jnp.where(col == 1, i2, 0)).astype(_i32)
    w_ref[...] = jnp.where(col == 0, w1, jnp.where(col == 1, w2, 0.0))


def _gate_body1(xa_ref, wg_ref, bg_ref, eid_ref, w_ref):
    logits = jnp.dot(xa_ref[...], wg_ref[...], preferred_element_type=_f32)
    logits += bg_ref[...]
    col = lax.broadcasted_iota(_i32, logits.shape, 1)
    i1, i2, w1, w2 = _top2(logits, col)
    eid_ref[...] = jnp.where(col == 0, i1, jnp.where(col == 1, i2, 0)).astype(_i32)
    w_ref[...] = jnp.where(col == 0, w1, jnp.where(col == 1, w2, 0.0))


def _gate(xa, xb, wg, bg):
    din = wg.shape[0]
    wg_pad = jnp.zeros((din, 128), _f32).at[:, :E].set(wg)
    bg_pad = jnp.zeros((1, 128), _f32).at[0, :E].set(bg)
    grid = (B // TMG,)
    row_spec = pl.BlockSpec((TMG, D0), lambda i: (i, 0))
    full = lambda shape: pl.BlockSpec(shape, lambda i: tuple(0 for _ in shape))
    out_specs = [pl.BlockSpec((TMG, 128), lambda i: (i, 0))] * 2
    out_shape = [jax.ShapeDtypeStruct((B, 128), _i32),
                 jax.ShapeDtypeStruct((B, 128), _f32)]
    if xb is None:
        eid, w = pl.pallas_call(
            _gate_body1, grid=grid,
            in_specs=[row_spec, full((din, 128)), full((1, 128))],
            out_specs=out_specs, out_shape=out_shape,
        )(xa, wg_pad, bg_pad)
    else:
        eid, w = pl.pallas_call(
            _gate_body2, grid=grid,
            in_specs=[row_spec, row_spec, full((din, 128)), full((1, 128))],
            out_specs=out_specs, out_shape=out_shape,
        )(xa, xb, wg_pad, bg_pad)
    return eid[:, :KTOP], w[:, :KTOP]


def _route(eid, w):
    ef = eid.reshape(-1).astype(_i32)
    wf = w.reshape(-1)
    onehot = (ef[:, None] == jnp.arange(E, dtype=_i32)[None, :]).astype(_i32)
    ranks = jnp.cumsum(onehot, axis=0) - onehot
    counts = ranks[-1] + onehot[-1]
    padded = ((counts + TM - 1) // TM) * TM
    off = jnp.concatenate([jnp.zeros((1,), _i32), jnp.cumsum(padded)[:-1].astype(_i32)])
    k = jnp.arange(NPAIR, dtype=_i32)
    dest = jnp.sum(jnp.where(onehot > 0, off[None, :] + ranks, 0), axis=1)
    src_tok = jnp.zeros((P,), _i32).at[dest].set(k // KTOP)
    ws = jnp.zeros((P,), _f32).at[dest].set(wf)
    t = jnp.arange(NT, dtype=_i32) * TM
    tile_eid = jnp.clip((t[:, None] >= off[None, :]).sum(1) - 1, 0, E - 1).astype(_i32)
    return src_tok, ws.reshape(P, 1), dest.reshape(B, KTOP), tile_eid


def _gather_rows(x, src_tok):
    return x[src_tok]


def _combine(ys, dest):
    return ys[dest[:, 0]] + ys[dest[:, 1]]


def _bf16r(x):
    return x.astype(jnp.bfloat16).astype(_f32)


def _gmm_body2(eid_ref, xa_ref, xb_ref, w_ref, b_ref, ws_ref, out_ref):
    acc = jnp.dot(xa_ref[...], w_ref[0, :D0, :], preferred_element_type=_f32)
    acc += jnp.dot(xb_ref[...], w_ref[0, D0:, :], preferred_element_type=_f32)
    acc += b_ref[0]
    acc = jnp.where(acc > 0, acc, 0.2 * acc)
    out_ref[...] = _bf16r(acc) * _bf16r(ws_ref[...])


def _gmm_body1(eid_ref, xa_ref, w_ref, b_ref, ws_ref, out_ref):
    acc = jnp.dot(xa_ref[...], w_ref[0], preferred_element_type=_f32)
    acc += b_ref[0]
    acc = jnp.where(acc > 0, acc, 0.2 * acc)
    out_ref[...] = _bf16r(acc) * _bf16r(ws_ref[...])


def _gmm(xa, xb, wexp, bexp, ws, tile_eid):
    din = wexp.shape[1]
    dout = wexp.shape[2]
    nn = dout // TN
    row = lambda j, i, eid: (i, 0)
    in_specs = [pl.BlockSpec((TM, D0), row)]
    args = [xa]
    if xb is not None:
        in_specs.append(pl.BlockSpec((TM, D0), row))
        args.append(xb)
    in_specs += [
        pl.BlockSpec((1, din, TN), lambda j, i, eid: (eid[i], 0, j)),
        pl.BlockSpec((1, 1, TN), lambda j, i, eid: (eid[i], 0, j)),
        pl.BlockSpec((TM, 1), row),
    ]
    args += [wexp, bexp.reshape(E, 1, dout), ws]
    grid_spec = pltpu.PrefetchScalarGridSpec(
        num_scalar_prefetch=1,
        grid=(nn, NT),
        in_specs=in_specs,
        out_specs=pl.BlockSpec((TM, TN), lambda j, i, eid: (i, j)),
    )
    body = _gmm_body1 if xb is None else _gmm_body2
    return pl.pallas_call(
        body, grid_spec=grid_spec,
        out_shape=jax.ShapeDtypeStruct((P, dout), _f32),
    )(tile_eid, *args)


def _final_body(xa_ref, xb_ref, wg_ref, bg_ref, wr_ref, br_ref, out_ref):
    logits = jnp.dot(xa_ref[...], wg_ref[:D0, :], preferred_element_type=_f32)
    logits += jnp.dot(xb_ref[...], wg_ref[D0:, :], preferred_element_type=_f32)
    logits += bg_ref[...]
    col = lax.broadcasted_iota(_i32, logits.shape, 1)
    i1, i2, w1, w2 = _top2(logits, col)
    h = jnp.dot(xa_ref[...], wr_ref[:D0, :], preferred_element_type=_f32)
    h += jnp.dot(xb_ref[...], wr_ref[D0:, :], preferred_element_type=_f32)
    h += br_ref[...]
    h1 = jnp.sum(jnp.where(col == i1, h, 0.0), axis=1, keepdims=True)
    h2 = jnp.sum(jnp.where(col == i2, h, 0.0), axis=1, keepdims=True)
    out = w1 * h1 + w2 * h2
    out_ref[...] = jnp.broadcast_to(out, out_ref.shape)


def _final(xa, xb, p):
    din = p['Wg'].shape[0]
    wg_pad = jnp.zeros((din, 128), _f32).at[:, :E].set(p['Wg'])
    bg_pad = jnp.zeros((1, 128), _f32).at[0, :E].set(p['bg'])
    wr_pad = jnp.zeros((din, 128), _f32).at[:, :E].set(p['W'][:, :, 0].T)
    br_pad = jnp.zeros((1, 128), _f32).at[0, :E].set(p['b'][:, 0])
    grid = (B // TMG,)
    row_spec = pl.BlockSpec((TMG, D0), lambda i: (i, 0))
    full = lambda shape: pl.BlockSpec(shape, lambda i: tuple(0 for _ in shape))
    out = pl.pallas_call(
        _final_body, grid=grid,
        in_specs=[row_spec, row_spec, full((din, 128)), full((1, 128)),
                  full((din, 128)), full((1, 128))],
        out_specs=pl.BlockSpec((TMG, 128), lambda i: (i, 0)),
        out_shape=jax.ShapeDtypeStruct((B, 128), _f32),
    )(xa, xb, wg_pad, bg_pad, wr_pad, br_pad)
    return out[:, :1]


def _layer(prev, x0, p):
    eid, w = _gate(x0 if prev is None else prev, None if prev is None else x0,
                   p['Wg'], p['bg'])
    src_tok, ws, dest, tile_eid = _route(eid, w)
    if prev is None:
        xa = _gather_rows(x0, src_tok)
        ys = _gmm(xa, None, p['W'], p['b'], ws, tile_eid)
    else:
        xa = _gather_rows(prev, src_tok)
        xb = _gather_rows(x0, src_tok)
        ys = _gmm(xa, xb, p['W'], p['b'], ws, tile_eid)
    return _combine(ys, dest)


@jax.jit
def kernel(states, actions, params):
    x0 = jnp.concatenate([states, actions], axis=-1)
    prev = None
    for l in range(4):
        prev = _layer(prev, x0, params['l%d' % l])
    return _final(prev, x0, params['l4'])


# speedup vs baseline: 1.1735x; 1.0613x over previous
import functools

import jax
import jax.numpy as jnp
from jax import lax
from jax.experimental import pallas as pl
from jax.experimental.pallas import tpu as pltpu

B = 4096
D0 = 1024
HID = 1024
E = 8
KTOP = 2
NPAIR = B * KTOP
TM = 128
P = NPAIR + E * TM
NT = P // TM
TN = 512
TMG = 512
NEG = -1e30

_f32 = jnp.float32
_i32 = jnp.int32


def _top2(logits, col):
    valid = col < E
    logits = jnp.where(valid, logits, NEG)
    m = jnp.max(logits, axis=1, keepdims=True)
    p = jnp.exp(logits - m)
    p = jnp.where(valid, p, 0.0)
    s = jnp.sum(p, axis=1, keepdims=True)
    g = p / s
    m1 = jnp.max(g, axis=1, keepdims=True)
    i1 = jnp.min(jnp.where(g == m1, col, 127), axis=1, keepdims=True)
    g2 = jnp.where(col == i1, -1.0, g)
    m2 = jnp.max(g2, axis=1, keepdims=True)
    i2 = jnp.min(jnp.where(g2 == m2, col, 127), axis=1, keepdims=True)
    denom = m1 + m2 + 1e-9
    return i1, i2, m1 / denom, m2 / denom


def _gate_body2(xa_ref, xb_ref, wg_ref, bg_ref, eid_ref, w_ref):
    logits = jnp.dot(xa_ref[...], wg_ref[:D0, :], preferred_element_type=_f32)
    logits += jnp.dot(xb_ref[...], wg_ref[D0:, :], preferred_element_type=_f32)
    logits += bg_ref[...]
    col = lax.broadcasted_iota(_i32, logits.shape, 1)
    i1, i2, w1, w2 = _top2(logits, col)
    eid_ref[...] = jnp.where(col == 0, i1, jnp.where(col == 1, i2, 0)).astype(_i32)
    w_ref[...] = jnp.where(col == 0, w1, jnp.where(col == 1, w2, 0.0))


def _gate_body1(xa_ref, wg_ref, bg_ref, eid_ref, w_ref):
    logits = jnp.dot(xa_ref[...], wg_ref[...], preferred_element_type=_f32)
    logits += bg_ref[...]
    col = lax.broadcasted_iota(_i32, logits.shape, 1)
    i1, i2, w1, w2 = _top2(logits, col)
    eid_ref[...] = jnp.where(col == 0, i1, jnp.where(col == 1, i2, 0)).astype(_i32)
    w_ref[...] = jnp.where(col == 0, w1, jnp.where(col == 1, w2, 0.0))


def _gate(xa, xb, wg, bg):
    din = wg.shape[0]
    wg_pad = jnp.zeros((din, 128), _f32).at[:, :E].set(wg)
    bg_pad = jnp.zeros((1, 128), _f32).at[0, :E].set(bg)
    grid = (B // TMG,)
    row_spec = pl.BlockSpec((TMG, D0), lambda i: (i, 0))
    full = lambda shape: pl.BlockSpec(shape, lambda i: tuple(0 for _ in shape))
    out_specs = [pl.BlockSpec((TMG, 128), lambda i: (i, 0))] * 2
    out_shape = [jax.ShapeDtypeStruct((B, 128), _i32),
                 jax.ShapeDtypeStruct((B, 128), _f32)]
    if xb is None:
        eid, w = pl.pallas_call(
            _gate_body1, grid=grid,
            in_specs=[row_spec, full((din, 128)), full((1, 128))],
            out_specs=out_specs, out_shape=out_shape,
        )(xa, wg_pad, bg_pad)
    else:
        eid, w = pl.pallas_call(
            _gate_body2, grid=grid,
            in_specs=[row_spec, row_spec, full((din, 128)), full((1, 128))],
            out_specs=out_specs, out_shape=out_shape,
        )(xa, xb, wg_pad, bg_pad)
    return eid[:, :KTOP], w[:, :KTOP]


def _route(eid, w):
    ef = eid.reshape(-1).astype(_i32)
    wf = w.reshape(-1)
    onehot = (ef[:, None] == jnp.arange(E, dtype=_i32)[None, :]).astype(_i32)
    ranks = jnp.cumsum(onehot, axis=0) - onehot
    counts = ranks[-1] + onehot[-1]
    padded = ((counts + TM - 1) // TM) * TM
    off = jnp.concatenate([jnp.zeros((1,), _i32), jnp.cumsum(padded)[:-1].astype(_i32)])
    dest = jnp.sum(jnp.where(onehot > 0, off[None, :] + ranks, 0), axis=1)
    t = jnp.arange(NT, dtype=_i32) * TM
    tile_eid = jnp.clip((t[:, None] >= off[None, :]).sum(1) - 1, 0, E - 1).astype(_i32)
    return (dest.reshape(_PAIR_ROWS, 128), wf.reshape(_PAIR_ROWS, 128),
            dest.reshape(B, KTOP), tile_eid)


def _gather_rows(x, src_tok):
    return x[src_tok]


_SC_CORES = 2
_SC_TILES = 16
_NW = _SC_CORES * _SC_TILES
_PAIR_ROWS = NPAIR // 128
_ROWS_PER_S = _PAIR_ROWS // _SC_TILES
_PPW = P // _NW
_ZPW = P // _SC_TILES
_GCH = 48
_NCH = _PPW // _GCH


def _make_route_gather(two_src):
    from jax.experimental.pallas import tpu_sc as plsc

    mesh = plsc.VectorSubcoreMesh(core_axis_name="c", subcore_axis_name="s")
    n_xs = 2 if two_src else 1
    out_type = ([jax.ShapeDtypeStruct((P, D0), _f32)] * n_xs
                + [jax.ShapeDtypeStruct((P,), _f32)])
    scratch = [
        pltpu.VMEM((_ROWS_PER_S, 128), _i32),
        pltpu.VMEM((_ROWS_PER_S, 128), _i32),
        pltpu.VMEM((_ROWS_PER_S, 128), _f32),
        pltpu.VMEM((_ZPW,), _i32),
        pltpu.VMEM((_ZPW,), _f32),
        pltpu.VMEM((_PPW,), _i32),
        pltpu.VMEM((_PPW,), _f32),
        pltpu.VMEM((_GCH, D0), _f32),
    ]
    if two_src:
        scratch.append(pltpu.VMEM((_GCH, D0), _f32))
    scratch += [
        pltpu.VMEM_SHARED((P,), _i32),
        pltpu.VMEM_SHARED((P,), _f32),
        pltpu.SemaphoreType.DMA,
        pltpu.SemaphoreType.DMA,
    ]

    def body(*refs):
        if two_src:
            (dest_hbm, w_hbm, srca, srcb, xsa, xsb, ws_out,
             dest_v, tok_v, wv, zi, zf, src_v, ws_v, rows_a, rows_b,
             src_sh, ws_sh, sem_a, sem_b) = refs
        else:
            (dest_hbm, w_hbm, srca, xsa, ws_out,
             dest_v, tok_v, wv, zi, zf, src_v, ws_v, rows_a,
             src_sh, ws_sh, sem_a, sem_b) = refs
        c = lax.axis_index("c")
        s = lax.axis_index("s")
        wid = s * _SC_CORES + c
        lane = lax.iota(_i32, 16)
        for i in range(_ZPW // 16):
            zi[pl.ds(i * 16, 16)] = jnp.zeros((16,), _i32)
            zf[pl.ds(i * 16, 16)] = jnp.zeros((16,), _f32)
        pltpu.sync_copy(zi, src_sh.at[pl.ds(s * _ZPW, _ZPW)])
        pltpu.sync_copy(zf, ws_sh.at[pl.ds(s * _ZPW, _ZPW)])
        r0 = s * _ROWS_PER_S
        pltpu.sync_copy(dest_hbm.at[pl.ds(r0, _ROWS_PER_S)], dest_v)
        pltpu.sync_copy(w_hbm.at[pl.ds(r0, _ROWS_PER_S)], wv)
        for r in range(_ROWS_PER_S):
            for i in range(8):
                j0 = (r0 + r) * 128 + i * 16
                tok_v[r, pl.ds(i * 16, 16)] = jnp.right_shift(lane + j0, 1)
        plsc.subcore_barrier()
        for r in range(_ROWS_PER_S):
            pltpu.sync_copy(tok_v.at[r], src_sh.at[dest_v.at[r]], add=True)
            pltpu.sync_copy(wv.at[r], ws_sh.at[dest_v.at[r]], add=True)
        plsc.subcore_barrier()
        p0 = wid * _PPW
        pltpu.sync_copy(src_sh.at[pl.ds(p0, _PPW)], src_v)
        pltpu.sync_copy(ws_sh.at[pl.ds(p0, _PPW)], ws_v)
        pltpu.sync_copy(ws_v, ws_out.at[pl.ds(p0, _PPW)])
        for ch in range(_NCH):
            idx = src_v.at[pl.ds(ch * _GCH, _GCH)]
            cpa = pltpu.async_copy(srca.at[idx], rows_a, sem_a)
            if two_src:
                cpb = pltpu.async_copy(srcb.at[idx], rows_b, sem_b)
            cpa.wait()
            pltpu.sync_copy(rows_a, xsa.at[pl.ds(p0 + ch * _GCH, _GCH)])
            if two_src:
                cpb.wait()
                pltpu.sync_copy(rows_b, xsb.at[pl.ds(p0 + ch * _GCH, _GCH)])

    return pl.kernel(body, out_type=out_type, mesh=mesh,
                     scratch_types=scratch)


_route_gather1 = _make_route_gather(False)
_route_gather2 = _make_route_gather(True)


def _combine(ys, dest):
    return ys[dest[:, 0]] + ys[dest[:, 1]]


def _bf16r(x):
    return x.astype(jnp.bfloat16).astype(_f32)


def _gmm_body2(eid_ref, xa_ref, xb_ref, w_ref, b_ref, ws_ref, out_ref):
    acc = jnp.dot(xa_ref[...], w_ref[0, :D0, :], preferred_element_type=_f32)
    acc += jnp.dot(xb_ref[...], w_ref[0, D0:, :], preferred_element_type=_f32)
    acc += b_ref[0]
    acc = jnp.where(acc > 0, acc, 0.2 * acc)
    out_ref[...] = _bf16r(acc) * _bf16r(ws_ref[...])


def _gmm_body1(eid_ref, xa_ref, w_ref, b_ref, ws_ref, out_ref):
    acc = jnp.dot(xa_ref[...], w_ref[0], preferred_element_type=_f32)
    acc += b_ref[0]
    acc = jnp.where(acc > 0, acc, 0.2 * acc)
    out_ref[...] = _bf16r(acc) * _bf16r(ws_ref[...])


def _gmm(xa, xb, wexp, bexp, ws, tile_eid):
    din = wexp.shape[1]
    dout = wexp.shape[2]
    nn = dout // TN
    row = lambda j, i, eid: (i, 0)
    in_specs = [pl.BlockSpec((TM, D0), row)]
    args = [xa]
    if xb is not None:
        in_specs.append(pl.BlockSpec((TM, D0), row))
        args.append(xb)
    in_specs += [
        pl.BlockSpec((1, din, TN), lambda j, i, eid: (eid[i], 0, j)),
        pl.BlockSpec((1, 1, TN), lambda j, i, eid: (eid[i], 0, j)),
        pl.BlockSpec((TM, 1), row),
    ]
    args += [wexp, bexp.reshape(E, 1, dout), ws]
    grid_spec = pltpu.PrefetchScalarGridSpec(
        num_scalar_prefetch=1,
        grid=(nn, NT),
        in_specs=in_specs,
        out_specs=pl.BlockSpec((TM, TN), lambda j, i, eid: (i, j)),
    )
    body = _gmm_body1 if xb is None else _gmm_body2
    return pl.pallas_call(
        body, grid_spec=grid_spec,
        out_shape=jax.ShapeDtypeStruct((P, dout), _f32),
    )(tile_eid, *args)


def _final_body(xa_ref, xb_ref, wg_ref, bg_ref, wr_ref, br_ref, out_ref):
    logits = jnp.dot(xa_ref[...], wg_ref[:D0, :], preferred_element_type=_f32)
    logits += jnp.dot(xb_ref[...], wg_ref[D0:, :], preferred_element_type=_f32)
    logits += bg_ref[...]
    col = lax.broadcasted_iota(_i32, logits.shape, 1)
    i1, i2, w1, w2 = _top2(logits, col)
    h = jnp.dot(xa_ref[...], wr_ref[:D0, :], preferred_element_type=_f32)
    h += jnp.dot(xb_ref[...], wr_ref[D0:, :], preferred_element_type=_f32)
    h += br_ref[...]
    h1 = jnp.sum(jnp.where(col == i1, h, 0.0), axis=1, keepdims=True)
    h2 = jnp.sum(jnp.where(col == i2, h, 0.0), axis=1, keepdims=True)
    out = w1 * h1 + w2 * h2
    out_ref[...] = jnp.broadcast_to(out, out_ref.shape)


def _final(xa, xb, p):
    din = p['Wg'].shape[0]
    wg_pad = jnp.zeros((din, 128), _f32).at[:, :E].set(p['Wg'])
    bg_pad = jnp.zeros((1, 128), _f32).at[0, :E].set(p['bg'])
    wr_pad = jnp.zeros((din, 128), _f32).at[:, :E].set(p['W'][:, :, 0].T)
    br_pad = jnp.zeros((1, 128), _f32).at[0, :E].set(p['b'][:, 0])
    grid = (B // TMG,)
    row_spec = pl.BlockSpec((TMG, D0), lambda i: (i, 0))
    full = lambda shape: pl.BlockSpec(shape, lambda i: tuple(0 for _ in shape))
    out = pl.pallas_call(
        _final_body, grid=grid,
        in_specs=[row_spec, row_spec, full((din, 128)), full((1, 128)),
                  full((din, 128)), full((1, 128))],
        out_specs=pl.BlockSpec((TMG, 128), lambda i: (i, 0)),
        out_shape=jax.ShapeDtypeStruct((B, 128), _f32),
    )(xa, xb, wg_pad, bg_pad, wr_pad, br_pad)
    return out[:, :1]


def _layer(prev, x0, p):
    eid, w = _gate(x0 if prev is None else prev, None if prev is None else x0,
                   p['Wg'], p['bg'])
    dest2d, w2d, dest, tile_eid = _route(eid, w)
    if prev is None:
        xa, ws = _route_gather1(dest2d, w2d, x0)
        ys = _gmm(xa, None, p['W'], p['b'], ws.reshape(P, 1), tile_eid)
    else:
        xa, xb, ws = _route_gather2(dest2d, w2d, prev, x0)
        ys = _gmm(xa, xb, p['W'], p['b'], ws.reshape(P, 1), tile_eid)
    return _combine(ys, dest)


@jax.jit
def kernel(states, actions, params):
    x0 = jnp.concatenate([states, actions], axis=-1)
    prev = None
    for l in range(4):
        prev = _layer(prev, x0, params['l%d' % l])
    return _final(prev, x0, params['l4'])


# speedup vs baseline: 1.1994x; 1.0221x over previous
import functools

import jax
import jax.numpy as jnp
from jax import lax
from jax.experimental import pallas as pl
from jax.experimental.pallas import tpu as pltpu

B = 4096
D0 = 1024
HID = 1024
E = 8
KTOP = 2
NPAIR = B * KTOP
TM = 128
P = NPAIR + E * TM
NT = P // TM
TN = 512
TMG = 512
NEG = -1e30

_f32 = jnp.float32
_i32 = jnp.int32


def _top2(logits, col):
    valid = col < E
    logits = jnp.where(valid, logits, NEG)
    m = jnp.max(logits, axis=1, keepdims=True)
    p = jnp.exp(logits - m)
    p = jnp.where(valid, p, 0.0)
    s = jnp.sum(p, axis=1, keepdims=True)
    g = p / s
    m1 = jnp.max(g, axis=1, keepdims=True)
    i1 = jnp.min(jnp.where(g == m1, col, 127), axis=1, keepdims=True)
    g2 = jnp.where(col == i1, -1.0, g)
    m2 = jnp.max(g2, axis=1, keepdims=True)
    i2 = jnp.min(jnp.where(g2 == m2, col, 127), axis=1, keepdims=True)
    denom = m1 + m2 + 1e-9
    return i1, i2, m1 / denom, m2 / denom


def _gate_body2(xa_ref, xb_ref, wg_ref, bg_ref, eid_ref, w_ref):
    logits = jnp.dot(xa_ref[...], wg_ref[:D0, :], preferred_element_type=_f32)
    logits += jnp.dot(xb_ref[...], wg_ref[D0:, :], preferred_element_type=_f32)
    logits += bg_ref[...]
    col = lax.broadcasted_iota(_i32, logits.shape, 1)
    i1, i2, w1, w2 = _top2(logits, col)
    eid_ref[...] = jnp.where(col == 0, i1, jnp.where(col == 1, i2, 0)).astype(_i32)
    w_ref[...] = jnp.where(col == 0, w1, jnp.where(col == 1, w2, 0.0))


def _gate_body1(xa_ref, wg_ref, bg_ref, eid_ref, w_ref):
    logits = jnp.dot(xa_ref[...], wg_ref[...], preferred_element_type=_f32)
    logits += bg_ref[...]
    col = lax.broadcasted_iota(_i32, logits.shape, 1)
    i1, i2, w1, w2 = _top2(logits, col)
    eid_ref[...] = jnp.where(col == 0, i1, jnp.where(col == 1, i2, 0)).astype(_i32)
    w_ref[...] = jnp.where(col == 0, w1, jnp.where(col == 1, w2, 0.0))


def _gate(xa, xb, wg, bg):
    din = wg.shape[0]
    wg_pad = jnp.zeros((din, 128), _f32).at[:, :E].set(wg)
    bg_pad = jnp.zeros((1, 128), _f32).at[0, :E].set(bg)
    grid = (B // TMG,)
    row_spec = pl.BlockSpec((TMG, D0), lambda i: (i, 0))
    full = lambda shape: pl.BlockSpec(shape, lambda i: tuple(0 for _ in shape))
    out_specs = [pl.BlockSpec((TMG, 128), lambda i: (i, 0))] * 2
    out_shape = [jax.ShapeDtypeStruct((B, 128), _i32),
                 jax.ShapeDtypeStruct((B, 128), _f32)]
    if xb is None:
        eid, w = pl.pallas_call(
            _gate_body1, grid=grid,
            in_specs=[row_spec, full((din, 128)), full((1, 128))],
            out_specs=out_specs, out_shape=out_shape,
        )(xa, wg_pad, bg_pad)
    else:
        eid, w = pl.pallas_call(
            _gate_body2, grid=grid,
            in_specs=[row_spec, row_spec, full((din, 128)), full((1, 128))],
            out_specs=out_specs, out_shape=out_shape,
        )(xa, xb, wg_pad, bg_pad)
    return eid[:, :KTOP], w[:, :KTOP]


def _route(eid, w):
    ef = eid.reshape(-1).astype(_i32)
    wf = w.reshape(-1)
    onehot = (ef[:, None] == jnp.arange(E, dtype=_i32)[None, :]).astype(_i32)
    ranks = jnp.cumsum(onehot, axis=0) - onehot
    counts = ranks[-1] + onehot[-1]
    padded = ((counts + TM - 1) // TM) * TM
    off = jnp.concatenate([jnp.zeros((1,), _i32), jnp.cumsum(padded)[:-1].astype(_i32)])
    dest = jnp.sum(jnp.where(onehot > 0, off[None, :] + ranks, 0), axis=1)
    t = jnp.arange(NT, dtype=_i32) * TM
    tile_eid = jnp.clip((t[:, None] >= off[None, :]).sum(1) - 1, 0, E - 1).astype(_i32)
    return (dest.reshape(_PAIR_ROWS, 128), wf.reshape(_PAIR_ROWS, 128),
            dest.reshape(B, KTOP), tile_eid)


def _gather_rows(x, src_tok):
    return x[src_tok]


_SC_CORES = 2
_SC_TILES = 16
_NW = _SC_CORES * _SC_TILES
_PAIR_ROWS = NPAIR // 128
_ROWS_PER_S = _PAIR_ROWS // _SC_TILES
_PPW = P // _NW
_ZPW = P // _SC_TILES
_GCH = 24
_NCH = _PPW // _GCH


def _make_route_gather(two_src):
    from jax.experimental.pallas import tpu_sc as plsc

    mesh = plsc.VectorSubcoreMesh(core_axis_name="c", subcore_axis_name="s")
    n_xs = 2 if two_src else 1
    out_type = ([jax.ShapeDtypeStruct((P, D0), _f32)] * n_xs
                + [jax.ShapeDtypeStruct((P,), _f32)])
    scratch = [
        pltpu.VMEM((_ROWS_PER_S, 128), _i32),
        pltpu.VMEM((_ROWS_PER_S, 128), _i32),
        pltpu.VMEM((_ROWS_PER_S, 128), _f32),
        pltpu.VMEM((_ZPW,), _i32),
        pltpu.VMEM((_ZPW,), _f32),
        pltpu.VMEM((_PPW,), _i32),
        pltpu.VMEM((_PPW,), _f32),
        pltpu.VMEM((_GCH, D0), _f32),
        pltpu.VMEM((_GCH, D0), _f32),
    ]
    if two_src:
        scratch += [pltpu.VMEM((_GCH, D0), _f32),
                    pltpu.VMEM((_GCH, D0), _f32)]
    scratch += [
        pltpu.VMEM_SHARED((P,), _i32),
        pltpu.VMEM_SHARED((P,), _f32),
        pltpu.SemaphoreType.DMA,
        pltpu.SemaphoreType.DMA,
        pltpu.SemaphoreType.DMA,
        pltpu.SemaphoreType.DMA,
    ]

    def body(*refs):
        if two_src:
            (dest_hbm, w_hbm, srca, srcb, xsa, xsb, ws_out,
             dest_v, tok_v, wv, zi, zf, src_v, ws_v, ra0, ra1, rb0, rb1,
             src_sh, ws_sh, sa0, sa1, sb0, sb1) = refs
            rows_a = (ra0, ra1)
            rows_b = (rb0, rb1)
            sem_a = (sa0, sa1)
            sem_b = (sb0, sb1)
        else:
            (dest_hbm, w_hbm, srca, xsa, ws_out,
             dest_v, tok_v, wv, zi, zf, src_v, ws_v, ra0, ra1,
             src_sh, ws_sh, sa0, sa1, sb0, sb1) = refs
            rows_a = (ra0, ra1)
            sem_a = (sa0, sa1)
        c = lax.axis_index("c")
        s = lax.axis_index("s")
        wid = s * _SC_CORES + c
        lane = lax.iota(_i32, 16)
        for i in range(_ZPW // 16):
            zi[pl.ds(i * 16, 16)] = jnp.zeros((16,), _i32)
            zf[pl.ds(i * 16, 16)] = jnp.zeros((16,), _f32)
        pltpu.sync_copy(zi, src_sh.at[pl.ds(s * _ZPW, _ZPW)])
        pltpu.sync_copy(zf, ws_sh.at[pl.ds(s * _ZPW, _ZPW)])
        r0 = s * _ROWS_PER_S
        pltpu.sync_copy(dest_hbm.at[pl.ds(r0, _ROWS_PER_S)], dest_v)
        pltpu.sync_copy(w_hbm.at[pl.ds(r0, _ROWS_PER_S)], wv)
        for r in range(_ROWS_PER_S):
            for i in range(8):
                j0 = (r0 + r) * 128 + i * 16
                tok_v[r, pl.ds(i * 16, 16)] = jnp.right_shift(lane + j0, 1)
        plsc.subcore_barrier()
        for r in range(_ROWS_PER_S):
            pltpu.sync_copy(tok_v.at[r], src_sh.at[dest_v.at[r]], add=True)
            pltpu.sync_copy(wv.at[r], ws_sh.at[dest_v.at[r]], add=True)
        plsc.subcore_barrier()
        p0 = wid * _PPW
        pltpu.sync_copy(src_sh.at[pl.ds(p0, _PPW)], src_v)
        pltpu.sync_copy(ws_sh.at[pl.ds(p0, _PPW)], ws_v)
        pltpu.sync_copy(ws_v, ws_out.at[pl.ds(p0, _PPW)])
        def issue(ch, b):
            idx = src_v.at[pl.ds(ch * _GCH, _GCH)]
            cpa = pltpu.async_copy(srca.at[idx], rows_a[b], sem_a[b])
            cpb = None
            if two_src:
                cpb = pltpu.async_copy(srcb.at[idx], rows_b[b], sem_b[b])
            return cpa, cpb

        pend = issue(0, 0)
        for ch in range(_NCH):
            b = ch & 1
            cpa, cpb = pend
            cpa.wait()
            if ch + 1 < _NCH:
                pend = issue(ch + 1, 1 - b)
            pltpu.sync_copy(rows_a[b], xsa.at[pl.ds(p0 + ch * _GCH, _GCH)])
            if two_src:
                cpb.wait()
                pltpu.sync_copy(rows_b[b], xsb.at[pl.ds(p0 + ch * _GCH, _GCH)])

    return pl.kernel(body, out_type=out_type, mesh=mesh,
                     scratch_types=scratch)


_route_gather1 = _make_route_gather(False)
_route_gather2 = _make_route_gather(True)


def _combine(ys, dest):
    return ys[dest[:, 0]] + ys[dest[:, 1]]


def _bf16r(x):
    return x.astype(jnp.bfloat16).astype(_f32)


def _gmm_body2(eid_ref, xa_ref, xb_ref, w_ref, b_ref, ws_ref, out_ref):
    acc = jnp.dot(xa_ref[...], w_ref[0, :D0, :], preferred_element_type=_f32)
    acc += jnp.dot(xb_ref[...], w_ref[0, D0:, :], preferred_element_type=_f32)
    acc += b_ref[0]
    acc = jnp.where(acc > 0, acc, 0.2 * acc)
    out_ref[...] = _bf16r(acc) * _bf16r(ws_ref[...])


def _gmm_body1(eid_ref, xa_ref, w_ref, b_ref, ws_ref, out_ref):
    acc = jnp.dot(xa_ref[...], w_ref[0], preferred_element_type=_f32)
    acc += b_ref[0]
    acc = jnp.where(acc > 0, acc, 0.2 * acc)
    out_ref[...] = _bf16r(acc) * _bf16r(ws_ref[...])


def _gmm(xa, xb, wexp, bexp, ws, tile_eid):
    din = wexp.shape[1]
    dout = wexp.shape[2]
    nn = dout // TN
    row = lambda j, i, eid: (i, 0)
    in_specs = [pl.BlockSpec((TM, D0), row)]
    args = [xa]
    if xb is not None:
        in_specs.append(pl.BlockSpec((TM, D0), row))
        args.append(xb)
    in_specs += [
        pl.BlockSpec((1, din, TN), lambda j, i, eid: (eid[i], 0, j)),
        pl.BlockSpec((1, 1, TN), lambda j, i, eid: (eid[i], 0, j)),
        pl.BlockSpec((TM, 1), row),
    ]
    args += [wexp, bexp.reshape(E, 1, dout), ws]
    grid_spec = pltpu.PrefetchScalarGridSpec(
        num_scalar_prefetch=1,
        grid=(nn, NT),
        in_specs=in_specs,
        out_specs=pl.BlockSpec((TM, TN), lambda j, i, eid: (i, j)),
    )
    body = _gmm_body1 if xb is None else _gmm_body2
    return pl.pallas_call(
        body, grid_spec=grid_spec,
        out_shape=jax.ShapeDtypeStruct((P, dout), _f32),
    )(tile_eid, *args)


def _final_body(xa_ref, xb_ref, wg_ref, bg_ref, wr_ref, br_ref, out_ref):
    logits = jnp.dot(xa_ref[...], wg_ref[:D0, :], preferred_element_type=_f32)
    logits += jnp.dot(xb_ref[...], wg_ref[D0:, :], preferred_element_type=_f32)
    logits += bg_ref[...]
    col = lax.broadcasted_iota(_i32, logits.shape, 1)
    i1, i2, w1, w2 = _top2(logits, col)
    h = jnp.dot(xa_ref[...], wr_ref[:D0, :], preferred_element_type=_f32)
    h += jnp.dot(xb_ref[...], wr_ref[D0:, :], preferred_element_type=_f32)
    h += br_ref[...]
    h1 = jnp.sum(jnp.where(col == i1, h, 0.0), axis=1, keepdims=True)
    h2 = jnp.sum(jnp.where(col == i2, h, 0.0), axis=1, keepdims=True)
    out = w1 * h1 + w2 * h2
    out_ref[...] = jnp.broadcast_to(out, out_ref.shape)


def _final(xa, xb, p):
    din = p['Wg'].shape[0]
    wg_pad = jnp.zeros((din, 128), _f32).at[:, :E].set(p['Wg'])
    bg_pad = jnp.zeros((1, 128), _f32).at[0, :E].set(p['bg'])
    wr_pad = jnp.zeros((din, 128), _f32).at[:, :E].set(p['W'][:, :, 0].T)
    br_pad = jnp.zeros((1, 128), _f32).at[0, :E].set(p['b'][:, 0])
    grid = (B // TMG,)
    row_spec = pl.BlockSpec((TMG, D0), lambda i: (i, 0))
    full = lambda shape: pl.BlockSpec(shape, lambda i: tuple(0 for _ in shape))
    out = pl.pallas_call(
        _final_body, grid=grid,
        in_specs=[row_spec, row_spec, full((din, 128)), full((1, 128)),
                  full((din, 128)), full((1, 128))],
        out_specs=pl.BlockSpec((TMG, 128), lambda i: (i, 0)),
        out_shape=jax.ShapeDtypeStruct((B, 128), _f32),
    )(xa, xb, wg_pad, bg_pad, wr_pad, br_pad)
    return out[:, :1]


def _layer(prev, x0, p):
    eid, w = _gate(x0 if prev is None else prev, None if prev is None else x0,
                   p['Wg'], p['bg'])
    dest2d, w2d, dest, tile_eid = _route(eid, w)
    if prev is None:
        xa, ws = _route_gather1(dest2d, w2d, x0)
        ys = _gmm(xa, None, p['W'], p['b'], ws.reshape(P, 1), tile_eid)
    else:
        xa, xb, ws = _route_gather2(dest2d, w2d, prev, x0)
        ys = _gmm(xa, xb, p['W'], p['b'], ws.reshape(P, 1), tile_eid)
    return _combine(ys, dest)


@jax.jit
def kernel(states, actions, params):
    x0 = jnp.concatenate([states, actions], axis=-1)
    prev = None
    for l in range(4):
        prev = _layer(prev, x0, params['l%d' % l])
    return _final(prev, x0, params['l4'])


# speedup vs baseline: 1.2792x; 1.0665x over previous
import functools

import jax
import jax.numpy as jnp
from jax import lax
from jax.experimental import pallas as pl
from jax.experimental.pallas import tpu as pltpu

B = 4096
D0 = 1024
HID = 1024
E = 8
KTOP = 2
NPAIR = B * KTOP
TM = 128
P = NPAIR + E * TM
NT = P // TM
TN = 512
TMG = 512
NEG = -1e30

_f32 = jnp.float32
_i32 = jnp.int32


def _top2(logits, col):
    valid = col < E
    logits = jnp.where(valid, logits, NEG)
    m = jnp.max(logits, axis=1, keepdims=True)
    p = jnp.exp(logits - m)
    p = jnp.where(valid, p, 0.0)
    s = jnp.sum(p, axis=1, keepdims=True)
    g = p / s
    m1 = jnp.max(g, axis=1, keepdims=True)
    i1 = jnp.min(jnp.where(g == m1, col, 127), axis=1, keepdims=True)
    g2 = jnp.where(col == i1, -1.0, g)
    m2 = jnp.max(g2, axis=1, keepdims=True)
    i2 = jnp.min(jnp.where(g2 == m2, col, 127), axis=1, keepdims=True)
    denom = m1 + m2 + 1e-9
    return i1, i2, m1 / denom, m2 / denom


def _gate_body2(xa_ref, xb_ref, wg_ref, bg_ref, eid_ref, w_ref):
    logits = jnp.dot(xa_ref[...], wg_ref[:D0, :], preferred_element_type=_f32)
    logits += jnp.dot(xb_ref[...], wg_ref[D0:, :], preferred_element_type=_f32)
    logits += bg_ref[...]
    col = lax.broadcasted_iota(_i32, logits.shape, 1)
    i1, i2, w1, w2 = _top2(logits, col)
    eid_ref[...] = jnp.where(col == 0, i1, jnp.where(col == 1, i2, 0)).astype(_i32)
    w_ref[...] = jnp.where(col == 0, w1, jnp.where(col == 1, w2, 0.0))


def _gate_body1(xa_ref, wg_ref, bg_ref, eid_ref, w_ref):
    logits = jnp.dot(xa_ref[...], wg_ref[...], preferred_element_type=_f32)
    logits += bg_ref[...]
    col = lax.broadcasted_iota(_i32, logits.shape, 1)
    i1, i2, w1, w2 = _top2(logits, col)
    eid_ref[...] = jnp.where(col == 0, i1, jnp.where(col == 1, i2, 0)).astype(_i32)
    w_ref[...] = jnp.where(col == 0, w1, jnp.where(col == 1, w2, 0.0))


def _gate(xa, xb, wg, bg):
    din = wg.shape[0]
    wg_pad = jnp.zeros((din, 128), _f32).at[:, :E].set(wg)
    bg_pad = jnp.zeros((1, 128), _f32).at[0, :E].set(bg)
    grid = (B // TMG,)
    row_spec = pl.BlockSpec((TMG, D0), lambda i: (i, 0))
    full = lambda shape: pl.BlockSpec(shape, lambda i: tuple(0 for _ in shape))
    out_specs = [pl.BlockSpec((TMG, 128), lambda i: (i, 0))] * 2
    out_shape = [jax.ShapeDtypeStruct((B, 128), _i32),
                 jax.ShapeDtypeStruct((B, 128), _f32)]
    if xb is None:
        eid, w = pl.pallas_call(
            _gate_body1, grid=grid,
            in_specs=[row_spec, full((din, 128)), full((1, 128))],
            out_specs=out_specs, out_shape=out_shape,
        )(xa, wg_pad, bg_pad)
    else:
        eid, w = pl.pallas_call(
            _gate_body2, grid=grid,
            in_specs=[row_spec, row_spec, full((din, 128)), full((1, 128))],
            out_specs=out_specs, out_shape=out_shape,
        )(xa, xb, wg_pad, bg_pad)
    return eid[:, :KTOP], w[:, :KTOP]


def _route(eid, w):
    ef = eid.reshape(-1).astype(_i32)
    wf = w.reshape(-1)
    onehot = (ef[:, None] == jnp.arange(E, dtype=_i32)[None, :]).astype(_i32)
    ranks = jnp.cumsum(onehot, axis=0) - onehot
    counts = ranks[-1] + onehot[-1]
    padded = ((counts + TM - 1) // TM) * TM
    off = jnp.concatenate([jnp.zeros((1,), _i32), jnp.cumsum(padded)[:-1].astype(_i32)])
    dest = jnp.sum(jnp.where(onehot > 0, off[None, :] + ranks, 0), axis=1)
    t = jnp.arange(NT, dtype=_i32) * TM
    tile_eid = jnp.clip((t[:, None] >= off[None, :]).sum(1) - 1, 0, E - 1).astype(_i32)
    return (dest.reshape(_PAIR_ROWS, 128), wf.reshape(_PAIR_ROWS, 128),
            dest.reshape(B, KTOP), tile_eid)


def _gather_rows(x, src_tok):
    return x[src_tok]


_SC_CORES = 2
_SC_TILES = 16
_NW = _SC_CORES * _SC_TILES
_PAIR_ROWS = NPAIR // 128
_ROWS_PER_S = _PAIR_ROWS // _SC_TILES
_PPW = P // _NW
_ZPW = P // _SC_TILES
_GCH = 24
_NCH = _PPW // _GCH


def _make_route_scatter():
    from jax.experimental.pallas import tpu_sc as plsc

    mesh = plsc.VectorSubcoreMesh(core_axis_name="c", subcore_axis_name="s")
    out_type = [jax.ShapeDtypeStruct((P,), _i32),
                jax.ShapeDtypeStruct((P,), _f32)]
    scratch = [
        pltpu.VMEM((_ROWS_PER_S, 128), _i32),
        pltpu.VMEM((_ROWS_PER_S, 128), _i32),
        pltpu.VMEM((_ROWS_PER_S, 128), _f32),
        pltpu.VMEM((_ZPW,), _i32),
        pltpu.VMEM((_ZPW,), _f32),
        pltpu.VMEM((_PPW,), _i32),
        pltpu.VMEM((_PPW,), _f32),
        pltpu.VMEM_SHARED((P,), _i32),
        pltpu.VMEM_SHARED((P,), _f32),
    ]

    def body(dest_hbm, w_hbm, src_out, ws_out,
             dest_v, tok_v, wv, zi, zf, src_v, ws_v, src_sh, ws_sh):
        c = lax.axis_index("c")
        s = lax.axis_index("s")
        wid = s * _SC_CORES + c
        lane = lax.iota(_i32, 16)
        for i in range(_ZPW // 16):
            zi[pl.ds(i * 16, 16)] = jnp.zeros((16,), _i32)
            zf[pl.ds(i * 16, 16)] = jnp.zeros((16,), _f32)
        pltpu.sync_copy(zi, src_sh.at[pl.ds(s * _ZPW, _ZPW)])
        pltpu.sync_copy(zf, ws_sh.at[pl.ds(s * _ZPW, _ZPW)])
        r0 = s * _ROWS_PER_S
        pltpu.sync_copy(dest_hbm.at[pl.ds(r0, _ROWS_PER_S)], dest_v)
        pltpu.sync_copy(w_hbm.at[pl.ds(r0, _ROWS_PER_S)], wv)
        for r in range(_ROWS_PER_S):
            for i in range(8):
                j0 = (r0 + r) * 128 + i * 16
                tok_v[r, pl.ds(i * 16, 16)] = jnp.right_shift(lane + j0, 1)
        plsc.subcore_barrier()
        for r in range(_ROWS_PER_S):
            pltpu.sync_copy(tok_v.at[r], src_sh.at[dest_v.at[r]], add=True)
            pltpu.sync_copy(wv.at[r], ws_sh.at[dest_v.at[r]], add=True)
        plsc.subcore_barrier()
        p0 = wid * _PPW
        pltpu.sync_copy(src_sh.at[pl.ds(p0, _PPW)], src_v)
        pltpu.sync_copy(ws_sh.at[pl.ds(p0, _PPW)], ws_v)
        pltpu.sync_copy(src_v, src_out.at[pl.ds(p0, _PPW)])
        pltpu.sync_copy(ws_v, ws_out.at[pl.ds(p0, _PPW)])

    return pl.kernel(body, out_type=out_type, mesh=mesh,
                     scratch_types=scratch)


_route_scatter = _make_route_scatter()


def _combine(ys, dest):
    return ys[dest[:, 0]] + ys[dest[:, 1]]


def _bf16r(x):
    return x.astype(jnp.bfloat16).astype(_f32)


def _gmm_body2(eid_ref, xa_ref, xb_ref, w_ref, b_ref, ws_ref, out_ref):
    acc = jnp.dot(xa_ref[...], w_ref[0, :D0, :], preferred_element_type=_f32)
    acc += jnp.dot(xb_ref[...], w_ref[0, D0:, :], preferred_element_type=_f32)
    acc += b_ref[0]
    acc = jnp.where(acc > 0, acc, 0.2 * acc)
    out_ref[...] = _bf16r(acc) * _bf16r(ws_ref[...])


def _gmm_body1(eid_ref, xa_ref, w_ref, b_ref, ws_ref, out_ref):
    acc = jnp.dot(xa_ref[...], w_ref[0], preferred_element_type=_f32)
    acc += b_ref[0]
    acc = jnp.where(acc > 0, acc, 0.2 * acc)
    out_ref[...] = _bf16r(acc) * _bf16r(ws_ref[...])


def _gmm(xa, xb, wexp, bexp, ws, tile_eid):
    din = wexp.shape[1]
    dout = wexp.shape[2]
    nn = dout // TN
    row = lambda j, i, eid: (i, 0)
    in_specs = [pl.BlockSpec((TM, D0), row)]
    args = [xa]
    if xb is not None:
        in_specs.append(pl.BlockSpec((TM, D0), row))
        args.append(xb)
    in_specs += [
        pl.BlockSpec((1, din, TN), lambda j, i, eid: (eid[i], 0, j)),
        pl.BlockSpec((1, 1, TN), lambda j, i, eid: (eid[i], 0, j)),
        pl.BlockSpec((TM, 1), row),
    ]
    args += [wexp, bexp.reshape(E, 1, dout), ws]
    grid_spec = pltpu.PrefetchScalarGridSpec(
        num_scalar_prefetch=1,
        grid=(nn, NT),
        in_specs=in_specs,
        out_specs=pl.BlockSpec((TM, TN), lambda j, i, eid: (i, j)),
    )
    body = _gmm_body1 if xb is None else _gmm_body2
    return pl.pallas_call(
        body, grid_spec=grid_spec,
        out_shape=jax.ShapeDtypeStruct((P, dout), _f32),
    )(tile_eid, *args)


def _final_body(xa_ref, xb_ref, wg_ref, bg_ref, wr_ref, br_ref, out_ref):
    logits = jnp.dot(xa_ref[...], wg_ref[:D0, :], preferred_element_type=_f32)
    logits += jnp.dot(xb_ref[...], wg_ref[D0:, :], preferred_element_type=_f32)
    logits += bg_ref[...]
    col = lax.broadcasted_iota(_i32, logits.shape, 1)
    i1, i2, w1, w2 = _top2(logits, col)
    h = jnp.dot(xa_ref[...], wr_ref[:D0, :], preferred_element_type=_f32)
    h += jnp.dot(xb_ref[...], wr_ref[D0:, :], preferred_element_type=_f32)
    h += br_ref[...]
    h1 = jnp.sum(jnp.where(col == i1, h, 0.0), axis=1, keepdims=True)
    h2 = jnp.sum(jnp.where(col == i2, h, 0.0), axis=1, keepdims=True)
    out = w1 * h1 + w2 * h2
    out_ref[...] = jnp.broadcast_to(out, out_ref.shape)


def _final(xa, xb, p):
    din = p['Wg'].shape[0]
    wg_pad = jnp.zeros((din, 128), _f32).at[:, :E].set(p['Wg'])
    bg_pad = jnp.zeros((1, 128), _f32).at[0, :E].set(p['bg'])
    wr_pad = jnp.zeros((din, 128), _f32).at[:, :E].set(p['W'][:, :, 0].T)
    br_pad = jnp.zeros((1, 128), _f32).at[0, :E].set(p['b'][:, 0])
    grid = (B // TMG,)
    row_spec = pl.BlockSpec((TMG, D0), lambda i: (i, 0))
    full = lambda shape: pl.BlockSpec(shape, lambda i: tuple(0 for _ in shape))
    out = pl.pallas_call(
        _final_body, grid=grid,
        in_specs=[row_spec, row_spec, full((din, 128)), full((1, 128)),
                  full((din, 128)), full((1, 128))],
        out_specs=pl.BlockSpec((TMG, 128), lambda i: (i, 0)),
        out_shape=jax.ShapeDtypeStruct((B, 128), _f32),
    )(xa, xb, wg_pad, bg_pad, wr_pad, br_pad)
    return out[:, :1]


def _layer(prev, x0, p):
    eid, w = _gate(x0 if prev is None else prev, None if prev is None else x0,
                   p['Wg'], p['bg'])
    dest2d, w2d, dest, tile_eid = _route(eid, w)
    src_tok, ws = _route_scatter(dest2d, w2d)
    if prev is None:
        ys = _gmm(x0[src_tok], None, p['W'], p['b'], ws.reshape(P, 1), tile_eid)
    else:
        ys = _gmm(prev[src_tok], x0[src_tok], p['W'], p['b'],
                  ws.reshape(P, 1), tile_eid)
    return _combine(ys, dest)


@jax.jit
def kernel(states, actions, params):
    x0 = jnp.concatenate([states, actions], axis=-1)
    prev = None
    for l in range(4):
        prev = _layer(prev, x0, params['l%d' % l])
    return _final(prev, x0, params['l4'])


# speedup vs baseline: 1.3163x; 1.0290x over previous
import functools

import jax
import jax.numpy as jnp
from jax import lax
from jax.experimental import pallas as pl
from jax.experimental.pallas import tpu as pltpu

B = 4096
D0 = 1024
HID = 1024
E = 8
KTOP = 2
NPAIR = B * KTOP
TM = 128
P = NPAIR + E * TM
NT = P // TM
TN = 512
TMG = 512
NEG = -1e30

_f32 = jnp.float32
_i32 = jnp.int32


def _top2(logits, col):
    valid = col < E
    logits = jnp.where(valid, logits, NEG)
    m = jnp.max(logits, axis=1, keepdims=True)
    p = jnp.exp(logits - m)
    p = jnp.where(valid, p, 0.0)
    s = jnp.sum(p, axis=1, keepdims=True)
    g = p / s
    m1 = jnp.max(g, axis=1, keepdims=True)
    i1 = jnp.min(jnp.where(g == m1, col, 127), axis=1, keepdims=True)
    g2 = jnp.where(col == i1, -1.0, g)
    m2 = jnp.max(g2, axis=1, keepdims=True)
    i2 = jnp.min(jnp.where(g2 == m2, col, 127), axis=1, keepdims=True)
    denom = m1 + m2 + 1e-9
    return i1, i2, m1 / denom, m2 / denom


def _gate_body2(xa_ref, xb_ref, wg_ref, bg_ref, eid_ref, w_ref):
    logits = jnp.dot(xa_ref[...], wg_ref[:D0, :], preferred_element_type=_f32)
    logits += jnp.dot(xb_ref[...], wg_ref[D0:, :], preferred_element_type=_f32)
    logits += bg_ref[...]
    col = lax.broadcasted_iota(_i32, logits.shape, 1)
    i1, i2, w1, w2 = _top2(logits, col)
    eid_ref[...] = jnp.where(col == 0, i1, jnp.where(col == 1, i2, 0)).astype(_i32)
    w_ref[...] = jnp.where(col == 0, w1, jnp.where(col == 1, w2, 0.0))


def _gate_body1(xa_ref, wg_ref, bg_ref, eid_ref, w_ref):
    logits = jnp.dot(xa_ref[...], wg_ref[...], preferred_element_type=_f32)
    logits += bg_ref[...]
    col = lax.broadcasted_iota(_i32, logits.shape, 1)
    i1, i2, w1, w2 = _top2(logits, col)
    eid_ref[...] = jnp.where(col == 0, i1, jnp.where(col == 1, i2, 0)).astype(_i32)
    w_ref[...] = jnp.where(col == 0, w1, jnp.where(col == 1, w2, 0.0))


def _gate(xa, xb, wg, bg):
    din = wg.shape[0]
    wg_pad = jnp.zeros((din, 128), _f32).at[:, :E].set(wg)
    bg_pad = jnp.zeros((1, 128), _f32).at[0, :E].set(bg)
    grid = (B // TMG,)
    row_spec = pl.BlockSpec((TMG, D0), lambda i: (i, 0))
    full = lambda shape: pl.BlockSpec(shape, lambda i: tuple(0 for _ in shape))
    out_specs = [pl.BlockSpec((TMG, 128), lambda i: (i, 0))] * 2
    out_shape = [jax.ShapeDtypeStruct((B, 128), _i32),
                 jax.ShapeDtypeStruct((B, 128), _f32)]
    if xb is None:
        eid, w = pl.pallas_call(
            _gate_body1, grid=grid,
            in_specs=[row_spec, full((din, 128)), full((1, 128))],
            out_specs=out_specs, out_shape=out_shape,
        )(xa, wg_pad, bg_pad)
    else:
        eid, w = pl.pallas_call(
            _gate_body2, grid=grid,
            in_specs=[row_spec, row_spec, full((din, 128)), full((1, 128))],
            out_specs=out_specs, out_shape=out_shape,
        )(xa, xb, wg_pad, bg_pad)
    return eid[:, :KTOP], w[:, :KTOP]


import numpy as _np

_TRI128 = jnp.asarray(_np.triu(_np.ones((128, 128), _np.float32)))
_LS64 = jnp.asarray(_np.tril(_np.ones((64, 64), _np.float32), k=-1))


def _routeidx_body(ef_ref, tri_ref, ls_ref, dest_ref, tile_ref):
    ef = ef_ref[...]
    tri = tri_ref[...]
    ls = ls_ref[...]
    dest = jnp.zeros(ef.shape, _f32)
    off = jnp.zeros((1, 1), _f32)
    offs = []
    for e in range(E):
        oh = (ef == e).astype(_f32)
        lanecum = jnp.dot(oh, tri, preferred_element_type=_f32)
        rowsum = lanecum[:, 127:128]
        rowpref = jnp.dot(ls, rowsum, preferred_element_type=_f32)
        rank = lanecum - oh + rowpref
        dest = dest + oh * (off + rank)
        offs.append(off)
        total = rowpref[63:64, :] + rowsum[63:64, :]
        padded = jnp.ceil(total / TM) * TM
        off = off + padded
    dest_ref[...] = dest.astype(_i32)
    pos = lax.broadcasted_iota(_i32, (1, 128), 1).astype(_f32) * TM
    te = jnp.full((1, 128), -1.0, _f32)
    for e in range(E):
        te += (pos >= offs[e]).astype(_f32)
    tile_ref[...] = jnp.clip(te, 0.0, E - 1.0).astype(_i32)


def _route(eid, w):
    ef2 = eid.reshape(_PAIR_ROWS, 128).astype(_i32)
    wf2 = w.reshape(_PAIR_ROWS, 128)
    full = lambda shape: pl.BlockSpec(shape, lambda: tuple(0 for _ in shape))
    dest2d, tile2d = pl.pallas_call(
        _routeidx_body, grid=(),
        in_specs=[full((_PAIR_ROWS, 128)), full((128, 128)), full((64, 64))],
        out_specs=[full((_PAIR_ROWS, 128)), full((1, 128))],
        out_shape=[jax.ShapeDtypeStruct((_PAIR_ROWS, 128), _i32),
                   jax.ShapeDtypeStruct((1, 128), _i32)],
    )(ef2, _TRI128, _LS64)
    return dest2d, wf2, dest2d.reshape(B, KTOP), tile2d[0, :NT]


def _gather_rows(x, src_tok):
    return x[src_tok]


_SC_CORES = 2
_SC_TILES = 16
_NW = _SC_CORES * _SC_TILES
_PAIR_ROWS = NPAIR // 128
_ROWS_PER_S = _PAIR_ROWS // _SC_TILES
_PPW = P // _NW
_ZPW = P // _SC_TILES
_GCH = 24
_NCH = _PPW // _GCH


def _make_route_scatter():
    from jax.experimental.pallas import tpu_sc as plsc

    mesh = plsc.VectorSubcoreMesh(core_axis_name="c", subcore_axis_name="s")
    out_type = [jax.ShapeDtypeStruct((P,), _i32),
                jax.ShapeDtypeStruct((P,), _f32)]
    scratch = [
        pltpu.VMEM((_ROWS_PER_S, 128), _i32),
        pltpu.VMEM((_ROWS_PER_S, 128), _i32),
        pltpu.VMEM((_ROWS_PER_S, 128), _f32),
        pltpu.VMEM((_ZPW,), _i32),
        pltpu.VMEM((_ZPW,), _f32),
        pltpu.VMEM((_PPW,), _i32),
        pltpu.VMEM((_PPW,), _f32),
        pltpu.VMEM_SHARED((P,), _i32),
        pltpu.VMEM_SHARED((P,), _f32),
    ]

    def body(dest_hbm, w_hbm, src_out, ws_out,
             dest_v, tok_v, wv, zi, zf, src_v, ws_v, src_sh, ws_sh):
        c = lax.axis_index("c")
        s = lax.axis_index("s")
        wid = s * _SC_CORES + c
        lane = lax.iota(_i32, 16)
        for i in range(_ZPW // 16):
            zi[pl.ds(i * 16, 16)] = jnp.zeros((16,), _i32)
            zf[pl.ds(i * 16, 16)] = jnp.zeros((16,), _f32)
        pltpu.sync_copy(zi, src_sh.at[pl.ds(s * _ZPW, _ZPW)])
        pltpu.sync_copy(zf, ws_sh.at[pl.ds(s * _ZPW, _ZPW)])
        r0 = s * _ROWS_PER_S
        pltpu.sync_copy(dest_hbm.at[pl.ds(r0, _ROWS_PER_S)], dest_v)
        pltpu.sync_copy(w_hbm.at[pl.ds(r0, _ROWS_PER_S)], wv)
        for r in range(_ROWS_PER_S):
            for i in range(8):
                j0 = (r0 + r) * 128 + i * 16
                tok_v[r, pl.ds(i * 16, 16)] = jnp.right_shift(lane + j0, 1)
        plsc.subcore_barrier()
        for r in range(_ROWS_PER_S):
            pltpu.sync_copy(tok_v.at[r], src_sh.at[dest_v.at[r]], add=True)
            pltpu.sync_copy(wv.at[r], ws_sh.at[dest_v.at[r]], add=True)
        plsc.subcore_barrier()
        p0 = wid * _PPW
        pltpu.sync_copy(src_sh.at[pl.ds(p0, _PPW)], src_v)
        pltpu.sync_copy(ws_sh.at[pl.ds(p0, _PPW)], ws_v)
        pltpu.sync_copy(src_v, src_out.at[pl.ds(p0, _PPW)])
        pltpu.sync_copy(ws_v, ws_out.at[pl.ds(p0, _PPW)])

    return pl.kernel(body, out_type=out_type, mesh=mesh,
                     scratch_types=scratch)


_route_scatter = _make_route_scatter()


def _combine(ys, dest):
    return ys[dest[:, 0]] + ys[dest[:, 1]]


def _bf16r(x):
    return x.astype(jnp.bfloat16).astype(_f32)


def _gmm_body2(eid_ref, xa_ref, xb_ref, w_ref, b_ref, ws_ref, out_ref):
    acc = jnp.dot(xa_ref[...], w_ref[0, :D0, :], preferred_element_type=_f32)
    acc += jnp.dot(xb_ref[...], w_ref[0, D0:, :], preferred_element_type=_f32)
    acc += b_ref[0]
    acc = jnp.where(acc > 0, acc, 0.2 * acc)
    out_ref[...] = _bf16r(acc) * _bf16r(ws_ref[...])


def _gmm_body1(eid_ref, xa_ref, w_ref, b_ref, ws_ref, out_ref):
    acc = jnp.dot(xa_ref[...], w_ref[0], preferred_element_type=_f32)
    acc += b_ref[0]
    acc = jnp.where(acc > 0, acc, 0.2 * acc)
    out_ref[...] = _bf16r(acc) * _bf16r(ws_ref[...])


def _gmm(xa, xb, wexp, bexp, ws, tile_eid):
    din = wexp.shape[1]
    dout = wexp.shape[2]
    nn = dout // TN
    row = lambda j, i, eid: (i, 0)
    in_specs = [pl.BlockSpec((TM, D0), row)]
    args = [xa]
    if xb is not None:
        in_specs.append(pl.BlockSpec((TM, D0), row))
        args.append(xb)
    in_specs += [
        pl.BlockSpec((1, din, TN), lambda j, i, eid: (eid[i], 0, j)),
        pl.BlockSpec((1, 1, TN), lambda j, i, eid: (eid[i], 0, j)),
        pl.BlockSpec((TM, 1), row),
    ]
    args += [wexp, bexp.reshape(E, 1, dout), ws]
    grid_spec = pltpu.PrefetchScalarGridSpec(
        num_scalar_prefetch=1,
        grid=(nn, NT),
        in_specs=in_specs,
        out_specs=pl.BlockSpec((TM, TN), lambda j, i, eid: (i, j)),
    )
    body = _gmm_body1 if xb is None else _gmm_body2
    return pl.pallas_call(
        body, grid_spec=grid_spec,
        out_shape=jax.ShapeDtypeStruct((P, dout), _f32),
    )(tile_eid, *args)


def _final_body(xa_ref, xb_ref, wg_ref, bg_ref, wr_ref, br_ref, out_ref):
    logits = jnp.dot(xa_ref[...], wg_ref[:D0, :], preferred_element_type=_f32)
    logits += jnp.dot(xb_ref[...], wg_ref[D0:, :], preferred_element_type=_f32)
    logits += bg_ref[...]
    col = lax.broadcasted_iota(_i32, logits.shape, 1)
    i1, i2, w1, w2 = _top2(logits, col)
    h = jnp.dot(xa_ref[...], wr_ref[:D0, :], preferred_element_type=_f32)
    h += jnp.dot(xb_ref[...], wr_ref[D0:, :], preferred_element_type=_f32)
    h += br_ref[...]
    h1 = jnp.sum(jnp.where(col == i1, h, 0.0), axis=1, keepdims=True)
    h2 = jnp.sum(jnp.where(col == i2, h, 0.0), axis=1, keepdims=True)
    out = w1 * h1 + w2 * h2
    out_ref[...] = jnp.broadcast_to(out, out_ref.shape)


def _final(xa, xb, p):
    din = p['Wg'].shape[0]
    wg_pad = jnp.zeros((din, 128), _f32).at[:, :E].set(p['Wg'])
    bg_pad = jnp.zeros((1, 128), _f32).at[0, :E].set(p['bg'])
    wr_pad = jnp.zeros((din, 128), _f32).at[:, :E].set(p['W'][:, :, 0].T)
    br_pad = jnp.zeros((1, 128), _f32).at[0, :E].set(p['b'][:, 0])
    grid = (B // TMG,)
    row_spec = pl.BlockSpec((TMG, D0), lambda i: (i, 0))
    full = lambda shape: pl.BlockSpec(shape, lambda i: tuple(0 for _ in shape))
    out = pl.pallas_call(
        _final_body, grid=grid,
        in_specs=[row_spec, row_spec, full((din, 128)), full((1, 128)),
                  full((din, 128)), full((1, 128))],
        out_specs=pl.BlockSpec((TMG, 128), lambda i: (i, 0)),
        out_shape=jax.ShapeDtypeStruct((B, 128), _f32),
    )(xa, xb, wg_pad, bg_pad, wr_pad, br_pad)
    return out[:, :1]


def _layer(prev, x0, p):
    eid, w = _gate(x0 if prev is None else prev, None if prev is None else x0,
                   p['Wg'], p['bg'])
    dest2d, w2d, dest, tile_eid = _route(eid, w)
    src_tok, ws = _route_scatter(dest2d, w2d)
    if prev is None:
        ys = _gmm(x0[src_tok], None, p['W'], p['b'], ws.reshape(P, 1), tile_eid)
    else:
        ys = _gmm(prev[src_tok], x0[src_tok], p['W'], p['b'],
                  ws.reshape(P, 1), tile_eid)
    return _combine(ys, dest)


@jax.jit
def kernel(states, actions, params):
    x0 = jnp.concatenate([states, actions], axis=-1)
    prev = None
    for l in range(4):
        prev = _layer(prev, x0, params['l%d' % l])
    return _final(prev, x0, params['l4'])
